# baseline (speedup 1.0000x reference)
import functools

import jax
import jax.numpy as jnp
from jax import lax
from jax.experimental import pallas as pl
from jax.experimental.pallas import tpu as pltpu
from jax.experimental.pallas import tpu_sc as plsc

D = 128
NW = 32
CHUNK = 128
EPS = 1e-5
NGRP = D // 16
UNROLL = 1


def _body(ids_hbm, table_hbm, pos_hbm, gamma_hbm, beta_hbm, out_hbm,
          idx_all, rows2, out2, pos_v, g_v, b_v, gsem, osem_a, osem_b):
    n_chunks = ids_hbm.shape[1]
    per_w = n_chunks * CHUNK
    l_seq = pos_v.shape[0]

    wid = lax.axis_index("s") * 2 + lax.axis_index("c")
    base = wid * per_w

    pltpu.sync_copy(ids_hbm.at[wid], idx_all)
    pltpu.sync_copy(pos_hbm, pos_v)
    pltpu.sync_copy(gamma_hbm, g_v)
    pltpu.sync_copy(beta_hbm, b_v)
    g = [g_v[pl.ds(16 * j, 16)] for j in range(NGRP)]
    b = [b_v[pl.ds(16 * j, 16)] for j in range(NGRP)]
    inv_d = 1.0 / D
    perms = [jnp.arange(16, dtype=jnp.int32) ^ k for k in (1, 2, 4, 8)]

    pltpu.async_copy(table_hbm.at[idx_all.at[0]], rows2.at[0], gsem)

    def chunk_body(t, carry):
        p = t % 2
        pn = (t + 1) % 2
        rbase = base + t * CHUNK
        pltpu.make_async_copy(table_hbm.at[idx_all.at[t]], rows2.at[p], gsem).wait()

        pltpu.async_copy(table_hbm.at[idx_all.at[(t + 1) % n_chunks]],
                         rows2.at[pn], gsem)

        @pl.when(jnp.logical_and(t >= 2, p == 0))
        def _drain_a():
            pltpu.make_async_copy(out2.at[p], out_hbm.at[pl.ds(rbase, CHUNK)],
                                  osem_a).wait()

        @pl.when(jnp.logical_and(t >= 2, p == 1))
        def _drain_b():
            pltpu.make_async_copy(out2.at[p], out_hbm.at[pl.ds(rbase, CHUNK)],
                                  osem_b).wait()

        poff = (t * CHUNK) % l_seq

        @plsc.parallel_loop(0, CHUNK, unroll=UNROLL)
        def _row(i):
            pi = poff + i
            pi = jnp.where(pi >= l_seq, pi - l_seq, pi)
            x = [rows2[p, i, pl.ds(16 * j, 16)] + pos_v[pi, pl.ds(16 * j, 16)]
                 for j in range(NGRP)]
            s = ((x[0] + x[1]) + (x[2] + x[3])) + ((x[4] + x[5]) + (x[6] + x[7]))
            sq = [xj * xj for xj in x]
            q = ((sq[0] + sq[1]) + (sq[2] + sq[3])) + ((sq[4] + sq[5]) + (sq[6] + sq[7]))
            for prm in perms:
                s = s + s.at[prm].get(mode="promise_in_bounds", unique_indices=True)
                q = q + q.at[prm].get(mode="promise_in_bounds", unique_indices=True)
            m = s * inv_d
            v = q * inv_d - m * m + EPS
            iv = lax.bitcast_convert_type(v, jnp.int32)
            magic = jnp.full((16,), 0x5F3759DF, dtype=jnp.int32)
            y = lax.bitcast_convert_type(
                magic - lax.shift_right_logical(iv, 1), jnp.float32)
            hv = 0.5 * v
            y = y * (1.5 - hv * y * y)
            y = y * (1.5 - hv * y * y)
            for j in range(NGRP):
                out2[p, i, pl.ds(16 * j, 16)] = (x[j] - m) * y * g[j] + b[j]

        @pl.when(p == 0)
        def _out_a():
            pltpu.async_copy(out2.at[p], out_hbm.at[pl.ds(rbase, CHUNK)], osem_a)

        @pl.when(p == 1)
        def _out_b():
            pltpu.async_copy(out2.at[p], out_hbm.at[pl.ds(rbase, CHUNK)], osem_b)

        return carry

    lax.fori_loop(0, n_chunks, chunk_body, 0)
    pltpu.make_async_copy(table_hbm.at[idx_all.at[0]], rows2.at[0], gsem).wait()
    pltpu.make_async_copy(out2.at[0], out_hbm.at[pl.ds(base, CHUNK)], osem_a).wait()
    pltpu.make_async_copy(out2.at[1], out_hbm.at[pl.ds(base, CHUNK)], osem_b).wait()


@jax.jit
def kernel(input_ids, token_table, pos_table, gamma, beta):
    bsz, l_seq = input_ids.shape
    n_tok = bsz * l_seq
    ids_flat = input_ids.reshape(NW, n_tok // CHUNK // NW, CHUNK)
    pos = pos_table[:l_seq]

    mesh = plsc.VectorSubcoreMesh(core_axis_name="c", subcore_axis_name="s")
    run = pl.kernel(
        _body,
        mesh=mesh,
        out_type=jax.ShapeDtypeStruct((n_tok, D), jnp.float32),
        scratch_types=[
            pltpu.VMEM((n_tok // CHUNK // NW, CHUNK), jnp.int32),
            pltpu.VMEM((2, CHUNK, D), jnp.float32),
            pltpu.VMEM((2, CHUNK, D), jnp.float32),
            pltpu.VMEM((l_seq, D), jnp.float32),
            pltpu.VMEM((D,), jnp.float32),
            pltpu.VMEM((D,), jnp.float32),
            pltpu.SemaphoreType.DMA,
            pltpu.SemaphoreType.DMA,
            pltpu.SemaphoreType.DMA,
        ],
    )
    out = run(ids_flat, token_table, pos, gamma, beta)
    return out.reshape(bsz, l_seq, D)


# speedup vs baseline: 1.2704x; 1.2704x over previous
import functools

import jax
import jax.numpy as jnp
from jax import lax
from jax.experimental import pallas as pl
from jax.experimental.pallas import tpu as pltpu
from jax.experimental.pallas import tpu_sc as plsc

D = 128
NW = 32
CHUNK = 128
EPS = 1e-5
NGRP = D // 16
UNROLL = 2


def _body(ids_hbm, table_hbm, pos_hbm, gamma_hbm, beta_hbm, out_hbm,
          idx_all, rows2, out2, pos_v, g_v, b_v, gsem, osem_a, osem_b):
    n_chunks = ids_hbm.shape[1]
    per_w = n_chunks * CHUNK
    l_seq = pos_v.shape[0]

    wid = lax.axis_index("s") * 2 + lax.axis_index("c")
    base = wid * per_w

    pltpu.sync_copy(ids_hbm.at[wid], idx_all)
    pltpu.sync_copy(pos_hbm, pos_v)
    pltpu.sync_copy(gamma_hbm, g_v)
    pltpu.sync_copy(beta_hbm, b_v)
    g = [g_v[pl.ds(16 * j, 16)] for j in range(NGRP)]
    b = [b_v[pl.ds(16 * j, 16)] for j in range(NGRP)]
    inv_d = 1.0 / D
    perms = [jnp.arange(16, dtype=jnp.int32) ^ k for k in (1, 2, 4, 8)]

    pltpu.async_copy(table_hbm.at[idx_all.at[0]], rows2.at[0], gsem)

    def chunk_body(t, carry):
        p = t % 2
        pn = (t + 1) % 2
        rbase = base + t * CHUNK
        pltpu.make_async_copy(table_hbm.at[idx_all.at[t]], rows2.at[p], gsem).wait()

        pltpu.async_copy(table_hbm.at[idx_all.at[(t + 1) % n_chunks]],
                         rows2.at[pn], gsem)

        @pl.when(jnp.logical_and(t >= 2, p == 0))
        def _drain_a():
            pltpu.make_async_copy(out2.at[p], out_hbm.at[pl.ds(rbase, CHUNK)],
                                  osem_a).wait()

        @pl.when(jnp.logical_and(t >= 2, p == 1))
        def _drain_b():
            pltpu.make_async_copy(out2.at[p], out_hbm.at[pl.ds(rbase, CHUNK)],
                                  osem_b).wait()

        poff = (t * CHUNK) % l_seq

        @plsc.parallel_loop(0, CHUNK, unroll=UNROLL)
        def _row(i):
            pi = poff + i
            pi = jnp.where(pi >= l_seq, pi - l_seq, pi)
            x = [rows2[p, i, pl.ds(16 * j, 16)] + pos_v[pi, pl.ds(16 * j, 16)]
                 for j in range(NGRP)]
            s = ((x[0] + x[1]) + (x[2] + x[3])) + ((x[4] + x[5]) + (x[6] + x[7]))
            sq = [xj * xj for xj in x]
            q = ((sq[0] + sq[1]) + (sq[2] + sq[3])) + ((sq[4] + sq[5]) + (sq[6] + sq[7]))
            for prm in perms:
                s = s + s.at[prm].get(mode="promise_in_bounds", unique_indices=True)
                q = q + q.at[prm].get(mode="promise_in_bounds", unique_indices=True)
            m = s * inv_d
            v = q * inv_d - m * m + EPS
            iv = lax.bitcast_convert_type(v, jnp.int32)
            magic = jnp.full((16,), 0x5F3759DF, dtype=jnp.int32)
            y = lax.bitcast_convert_type(
                magic - lax.shift_right_logical(iv, 1), jnp.float32)
            hv = 0.5 * v
            y = y * (1.5 - hv * y * y)
            for j in range(NGRP):
                out2[p, i, pl.ds(16 * j, 16)] = (x[j] - m) * y * g[j] + b[j]

        @pl.when(p == 0)
        def _out_a():
            pltpu.async_copy(out2.at[p], out_hbm.at[pl.ds(rbase, CHUNK)], osem_a)

        @pl.when(p == 1)
        def _out_b():
            pltpu.async_copy(out2.at[p], out_hbm.at[pl.ds(rbase, CHUNK)], osem_b)

        return carry

    lax.fori_loop(0, n_chunks, chunk_body, 0)
    pltpu.make_async_copy(table_hbm.at[idx_all.at[0]], rows2.at[0], gsem).wait()
    pltpu.make_async_copy(out2.at[0], out_hbm.at[pl.ds(base, CHUNK)], osem_a).wait()
    pltpu.make_async_copy(out2.at[1], out_hbm.at[pl.ds(base, CHUNK)], osem_b).wait()


@jax.jit
def kernel(input_ids, token_table, pos_table, gamma, beta):
    bsz, l_seq = input_ids.shape
    n_tok = bsz * l_seq
    ids_flat = input_ids.reshape(NW, n_tok // CHUNK // NW, CHUNK)
    pos = pos_table[:l_seq]

    mesh = plsc.VectorSubcoreMesh(core_axis_name="c", subcore_axis_name="s")
    run = pl.kernel(
        _body,
        mesh=mesh,
        out_type=jax.ShapeDtypeStruct((n_tok, D), jnp.float32),
        scratch_types=[
            pltpu.VMEM((n_tok // CHUNK // NW, CHUNK), jnp.int32),
            pltpu.VMEM((2, CHUNK, D), jnp.float32),
            pltpu.VMEM((2, CHUNK, D), jnp.float32),
            pltpu.VMEM((l_seq, D), jnp.float32),
            pltpu.VMEM((D,), jnp.float32),
            pltpu.VMEM((D,), jnp.float32),
            pltpu.SemaphoreType.DMA,
            pltpu.SemaphoreType.DMA,
            pltpu.SemaphoreType.DMA,
        ],
    )
    out = run(ids_flat, token_table, pos, gamma, beta)
    return out.reshape(bsz, l_seq, D)


# speedup vs baseline: 1.2709x; 1.0003x over previous
import jax
import jax.numpy as jnp
from jax import lax
from jax.experimental import pallas as pl
from jax.experimental.pallas import tpu as pltpu
from jax.experimental.pallas import tpu_sc as plsc

D = 128
NW = 32
CHUNK = 128
EPS = 1e-5
NGRP = D // 16
UNROLL = 2


def _body(ids_hbm, table_hbm, pos_hbm, gamma_hbm, beta_hbm, out_hbm,
          idx_all, rows2, out2, pos_v, g_v, b_v, gsem, osem_a, osem_b):
    n_chunks = ids_hbm.shape[1]
    per_w = n_chunks * CHUNK
    l_seq = pos_v.shape[0]

    wid = lax.axis_index("s") * 2 + lax.axis_index("c")
    base = wid * per_w

    pltpu.sync_copy(ids_hbm.at[wid], idx_all)
    pltpu.sync_copy(pos_hbm, pos_v)
    pltpu.sync_copy(gamma_hbm, g_v)
    pltpu.sync_copy(beta_hbm, b_v)
    g = [g_v[pl.ds(16 * j, 16)] for j in range(NGRP)]
    b = [b_v[pl.ds(16 * j, 16)] for j in range(NGRP)]
    inv_d = 1.0 / D
    perms = [jnp.arange(16, dtype=jnp.int32) ^ k for k in (1, 2, 4, 8)]

    pltpu.async_copy(table_hbm.at[idx_all.at[0]], rows2.at[0], gsem)

    def chunk_body(t, carry):
        p = t % 2
        pn = (t + 1) % 2
        rbase = base + t * CHUNK
        pltpu.make_async_copy(table_hbm.at[idx_all.at[t]], rows2.at[p], gsem).wait()

        pltpu.async_copy(table_hbm.at[idx_all.at[(t + 1) % n_chunks]],
                         rows2.at[pn], gsem)

        @pl.when(jnp.logical_and(t >= 2, p == 0))
        def _drain_a():
            pltpu.make_async_copy(out2.at[p], out_hbm.at[pl.ds(rbase, CHUNK)],
                                  osem_a).wait()

        @pl.when(jnp.logical_and(t >= 2, p == 1))
        def _drain_b():
            pltpu.make_async_copy(out2.at[p], out_hbm.at[pl.ds(rbase, CHUNK)],
                                  osem_b).wait()

        poff = (t * CHUNK) % l_seq

        @plsc.parallel_loop(0, CHUNK, unroll=UNROLL)
        def _row(i):
            pi = poff + i
            pi = jnp.where(pi >= l_seq, pi - l_seq, pi)
            x = [rows2[p, i, pl.ds(16 * j, 16)] + pos_v[pi, pl.ds(16 * j, 16)]
                 for j in range(NGRP)]
            s = ((x[0] + x[1]) + (x[2] + x[3])) + ((x[4] + x[5]) + (x[6] + x[7]))
            sq = [xj * xj for xj in x]
            q = ((sq[0] + sq[1]) + (sq[2] + sq[3])) + ((sq[4] + sq[5]) + (sq[6] + sq[7]))
            for prm in perms:
                s = s + s.at[prm].get(mode="promise_in_bounds", unique_indices=True)
                q = q + q.at[prm].get(mode="promise_in_bounds", unique_indices=True)
            m = s * inv_d
            v = q * inv_d - m * m + EPS
            iv = lax.bitcast_convert_type(v, jnp.int32)
            magic = jnp.full((16,), 0x5F3759DF, dtype=jnp.int32)
            y = lax.bitcast_convert_type(
                magic - lax.shift_right_logical(iv, 1), jnp.float32)
            hv = 0.5 * v
            y = y * (1.5 - hv * y * y)
            for j in range(NGRP):
                out2[p, i, pl.ds(16 * j, 16)] = (x[j] - m) * y * g[j] + b[j]

        @pl.when(p == 0)
        def _out_a():
            pltpu.async_copy(out2.at[p], out_hbm.at[pl.ds(rbase, CHUNK)], osem_a)

        @pl.when(p == 1)
        def _out_b():
            pltpu.async_copy(out2.at[p], out_hbm.at[pl.ds(rbase, CHUNK)], osem_b)

        return carry

    lax.fori_loop(0, n_chunks, chunk_body, 0)
    pltpu.make_async_copy(table_hbm.at[idx_all.at[0]], rows2.at[0], gsem).wait()
    pltpu.make_async_copy(out2.at[0], out_hbm.at[pl.ds(base, CHUNK)], osem_a).wait()
    pltpu.make_async_copy(out2.at[1], out_hbm.at[pl.ds(base, CHUNK)], osem_b).wait()


@jax.jit
def kernel(input_ids, token_table, pos_table, gamma, beta):
    bsz, l_seq = input_ids.shape
    n_tok = bsz * l_seq
    ids_flat = input_ids.reshape(NW, n_tok // CHUNK // NW, CHUNK)
    pos = pos_table[:l_seq]

    mesh = plsc.VectorSubcoreMesh(core_axis_name="c", subcore_axis_name="s")
    run = pl.kernel(
        _body,
        mesh=mesh,
        out_type=jax.ShapeDtypeStruct((n_tok, D), jnp.float32),
        scratch_types=[
            pltpu.VMEM((n_tok // CHUNK // NW, CHUNK), jnp.int32),
            pltpu.VMEM((2, CHUNK, D), jnp.float32),
            pltpu.VMEM((2, CHUNK, D), jnp.float32),
            pltpu.VMEM((l_seq, D), jnp.float32),
            pltpu.VMEM((D,), jnp.float32),
            pltpu.VMEM((D,), jnp.float32),
            pltpu.SemaphoreType.DMA,
            pltpu.SemaphoreType.DMA,
            pltpu.SemaphoreType.DMA,
        ],
    )
    out = run(ids_flat, token_table, pos, gamma, beta)
    return out.reshape(bsz, l_seq, D)
